# Initial kernel scaffold; baseline (speedup 1.0000x reference)
import jax
import jax.numpy as jnp
from jax.experimental import pallas as pl


def kernel(h_identity, h_others, W_quant_id, b_quant_id, codebook_id, W_post_id, b_post_id, W_quant_oth, b_quant_oth, codebook_oth, W_post_oth, b_post_oth):
    raise NotImplementedError("write your pallas kernel here")


# speedup vs baseline: 1.4794x; 1.4794x over previous
import functools

import jax
import jax.numpy as jnp
from jax.experimental import pallas as pl
from jax.experimental.pallas import tpu as pltpu

_BETA = 0.25


def _vq_branch_body(hf_ref, wqT_ref, bq_ref, emb_ref, embT_ref, wpT_ref, bp_ref,
                    out_ref, loss_ref, *, blk, K):
    z = jnp.dot(hf_ref[...], wqT_ref[...],
                preferred_element_type=jnp.float32) + bq_ref[...]
    ab = jnp.dot(z, embT_ref[...], preferred_element_type=jnp.float32)
    z2 = jnp.sum(z * z, axis=1, keepdims=True)
    e2 = jnp.sum(emb_ref[...] * emb_ref[...], axis=1)
    d = z2 + e2[None, :] - 2.0 * ab
    minval = jnp.min(d, axis=1)
    iota = jax.lax.broadcasted_iota(jnp.int32, (blk, K), 1)
    idx = jnp.min(jnp.where(d <= minval[:, None], iota, K), axis=1)
    oh = (iota == idx[:, None]).astype(jnp.float32)
    zq = jnp.dot(oh, emb_ref[...], preferred_element_type=jnp.float32)
    out_ref[...] = jnp.dot(zq, wpT_ref[...],
                           preferred_element_type=jnp.float32) + bp_ref[...]
    partial = jnp.sum(minval).reshape(1, 1)

    @pl.when(pl.program_id(0) == 0)
    def _init():
        loss_ref[...] = partial

    @pl.when(pl.program_id(0) != 0)
    def _acc():
        loss_ref[...] += partial


def _vq_branch(hf, Wq, bq, emb, Wp, bp, blk=512):
    N, Cin = hf.shape
    D = Wq.shape[0]
    K = emb.shape[0]
    Cout = Wp.shape[0]
    grid = N // blk
    out, loss_sum = pl.pallas_call(
        functools.partial(_vq_branch_body, blk=blk, K=K),
        grid=(grid,),
        in_specs=[
            pl.BlockSpec((blk, Cin), lambda i: (i, 0)),
            pl.BlockSpec((Cin, D), lambda i: (0, 0)),
            pl.BlockSpec((1, D), lambda i: (0, 0)),
            pl.BlockSpec((K, D), lambda i: (0, 0)),
            pl.BlockSpec((D, K), lambda i: (0, 0)),
            pl.BlockSpec((D, Cout), lambda i: (0, 0)),
            pl.BlockSpec((1, Cout), lambda i: (0, 0)),
        ],
        out_specs=[
            pl.BlockSpec((blk, Cout), lambda i: (i, 0)),
            pl.BlockSpec((1, 1), lambda i: (0, 0)),
        ],
        out_shape=[
            jax.ShapeDtypeStruct((N, Cout), jnp.float32),
            jax.ShapeDtypeStruct((1, 1), jnp.float32),
        ],
    )(hf, Wq.T, bq[None, :], emb, emb.T, Wp.T, bp[None, :])
    loss = (1.0 + _BETA) * loss_sum[0, 0] / (N * D)
    return out, loss


def kernel(h_identity, h_others, W_quant_id, b_quant_id, codebook_id,
           W_post_id, b_post_id, W_quant_oth, b_quant_oth, codebook_oth,
           W_post_oth, b_post_oth):
    B, C_id, H, W = h_identity.shape
    C_oth = h_others.shape[1]
    N = B * H * W
    hf_id = h_identity.transpose(0, 2, 3, 1).reshape(N, C_id)
    hf_oth = h_others.transpose(0, 2, 3, 1).reshape(N, C_oth)

    out_id, loss_id = _vq_branch(hf_id, W_quant_id, b_quant_id, codebook_id,
                                 W_post_id, b_post_id)
    out_oth, loss_oth = _vq_branch(hf_oth, W_quant_oth, b_quant_oth,
                                   codebook_oth, W_post_oth, b_post_oth)

    out_id = out_id.reshape(B, H, W, C_id).transpose(0, 3, 1, 2)
    out_oth = out_oth.reshape(B, H, W, C_oth).transpose(0, 3, 1, 2)
    out = jnp.concatenate([out_id, out_oth], axis=1)
    return out, loss_id + loss_oth
